# baseline (speedup 1.0000x reference)
import functools

import jax
import jax.numpy as jnp
from jax import lax
from jax.experimental import pallas as pl
from jax.experimental.pallas import tpu as pltpu
from jax.experimental.pallas import tpu_sc as plsc

DIM = 768
LANES = 16
NJ = DIM // LANES
EPS = 1e-12

NC = 2
NS = 16
NW = NC * NS

TOKENS = 8192
TPW = TOKENS // NW
CHUNK = 64
NCHUNK = TPW // CHUNK


def _xlane_sum(v):
    for sh in (1, 2, 4, 8):
        perm = lax.iota(jnp.int32, LANES) ^ sh
        pv = lax.gather(
            v, perm[:, None],
            lax.GatherDimensionNumbers(
                offset_dims=(), collapsed_slice_dims=(0,), start_index_map=(0,)),
            slice_sizes=(1,),
            mode=lax.GatherScatterMode.PROMISE_IN_BOUNDS)
        v = v + pv
    return v


def _rsqrt_vec(x):
    i = lax.bitcast_convert_type(x, jnp.int32)
    i = jnp.int32(0x5F3759DF) - lax.shift_right_logical(i, 1)
    y = lax.bitcast_convert_type(i, jnp.float32)
    for _ in range(3):
        y = y * (1.5 - 0.5 * x * y * y)
    return y


def _body(ids_hbm, pids_hbm, tok_hbm, pos_hbm, gamma_hbm, beta_hbm, out_hbm,
          idx_t, idx_p, buf_a, buf_b, gvec, bvec, sem_a, sem_b):
    wid = lax.axis_index("s") * NC + lax.axis_index("c")
    base = wid * TPW

    pltpu.sync_copy(gamma_hbm, gvec)
    pltpu.sync_copy(beta_hbm, bvec)

    def chunk_body(g, carry):
        cbase = base + g * CHUNK
        pltpu.sync_copy(ids_hbm.at[pl.ds(cbase, CHUNK)], idx_t)
        pltpu.sync_copy(pids_hbm.at[pl.ds(cbase, CHUNK)], idx_p)
        cp_a = pltpu.async_copy(tok_hbm.at[idx_t], buf_a, sem_a)
        cp_b = pltpu.async_copy(pos_hbm.at[idx_p], buf_b, sem_b)
        cp_a.wait()
        cp_b.wait()

        def tok_body(t, tcarry):
            s = jnp.zeros((LANES,), jnp.float32)
            q = jnp.zeros((LANES,), jnp.float32)
            for j in range(NJ):
                sl = pl.ds(j * LANES, LANES)
                v = buf_a[t, sl] + buf_b[t, sl]
                buf_a[t, sl] = v
                s = s + v
                q = q + v * v
            mvec = _xlane_sum(s) * (1.0 / DIM)
            var = _xlane_sum(q) * (1.0 / DIM) - mvec * mvec
            rvec = _rsqrt_vec(var + EPS)
            for j in range(NJ):
                sl = pl.ds(j * LANES, LANES)
                y = (buf_a[t, sl] - mvec) * rvec
                buf_a[t, sl] = y * gvec[sl] + bvec[sl]
            return tcarry

        lax.fori_loop(0, CHUNK, tok_body, 0)
        pltpu.sync_copy(buf_a, out_hbm.at[pl.ds(cbase, CHUNK)])
        return carry

    lax.fori_loop(0, NCHUNK, chunk_body, 0)


@jax.jit
def _sc_embed_ln(ids, pids, tok_emb, pos_emb, gamma, beta):
    mesh = plsc.VectorSubcoreMesh(
        core_axis_name="c", subcore_axis_name="s", num_cores=NC, num_subcores=NS)
    return pl.kernel(
        _body,
        out_type=jax.ShapeDtypeStruct((TOKENS, DIM), jnp.float32),
        mesh=mesh,
        scratch_types=[
            pltpu.VMEM((CHUNK,), jnp.int32),
            pltpu.VMEM((CHUNK,), jnp.int32),
            pltpu.VMEM((CHUNK, DIM), jnp.float32),
            pltpu.VMEM((CHUNK, DIM), jnp.float32),
            pltpu.VMEM((DIM,), jnp.float32),
            pltpu.VMEM((DIM,), jnp.float32),
            pltpu.SemaphoreType.DMA,
            pltpu.SemaphoreType.DMA,
        ],
    )(ids, pids, tok_emb, pos_emb, gamma, beta)


def kernel(input_ids, positional_ids, tok_emb, pos_emb, gamma, beta):
    ids = input_ids.reshape(-1).astype(jnp.int32)
    pids = positional_ids.reshape(-1).astype(jnp.int32)
    out = _sc_embed_ln(ids, pids, tok_emb, pos_emb, gamma, beta)
    return out.reshape(input_ids.shape + (DIM,))
